# initial kernel scaffold (unmeasured)
import jax
import jax.numpy as jnp
from jax import lax
from jax.experimental import pallas as pl
from jax.experimental.pallas import tpu as pltpu

Y = 4
NEG = -1e30


def kernel(x, W, labels):
    T, D = x.shape
    Vs = W.shape[1]
    labels2 = labels.reshape(T, 1)

    def body(x_ref, w_ref, lab_ref, out_ref,
             stats_ref, comm_ref, send_sems, recv_sems):
        my_x = lax.axis_index("x")
        my_y = lax.axis_index("y")
        my_z = lax.axis_index("z")

        barrier = pltpu.get_barrier_semaphore()
        for d in range(1, Y):
            pl.semaphore_signal(
                barrier,
                inc=1,
                device_id=(my_x, (my_y + d) % Y, my_z),
                device_id_type=pl.DeviceIdType.MESH,
            )
        pl.semaphore_wait(barrier, Y - 1)

        xb = x_ref[...].astype(jnp.bfloat16)
        wb = w_ref[...].astype(jnp.bfloat16)
        logits = jnp.dot(xb, wb, preferred_element_type=jnp.float32)
        m = jnp.max(logits, axis=1)
        s = jnp.sum(jnp.exp(logits - m[:, None]), axis=1)
        cols = lax.broadcasted_iota(jnp.int32, (T, Vs), 1) + my_y * Vs
        mask = cols == lab_ref[...]
        lab = jnp.max(jnp.where(mask, logits, NEG), axis=1)

        stats_ref[0, :] = m
        stats_ref[1, :] = s
        stats_ref[2, :] = lab

        send_rdmas = []
        for d in range(1, Y):
            slot = Y - d - 1
            rdma = pltpu.make_async_remote_copy(
                src_ref=stats_ref,
                dst_ref=comm_ref.at[slot],
                send_sem=send_sems.at[d - 1],
                recv_sem=recv_sems.at[slot],
                device_id=(my_x, (my_y + d) % Y, my_z),
                device_id_type=pl.DeviceIdType.MESH,
            )
            rdma.start()
            send_rdmas.append(rdma)

        for k in range(Y - 1):
            pltpu.make_async_remote_copy(
                src_ref=stats_ref,
                dst_ref=comm_ref.at[k],
                send_sem=send_sems.at[0],
                recv_sem=recv_sems.at[k],
                device_id=(my_x, my_y, my_z),
                device_id_type=pl.DeviceIdType.MESH,
            ).wait_recv()

        gm = m
        for k in range(Y - 1):
            gm = jnp.maximum(gm, comm_ref[k, 0, :])
        stot = s * jnp.exp(m - gm)
        for k in range(Y - 1):
            stot = stot + comm_ref[k, 1, :] * jnp.exp(comm_ref[k, 0, :] - gm)
        glab = lab
        for k in range(Y - 1):
            glab = jnp.maximum(glab, comm_ref[k, 2, :])
        out_ref[...] = gm + jnp.log(stot) - glab

        for rdma in send_rdmas:
            rdma.wait_send()

    return pl.pallas_call(
        body,
        out_shape=jax.ShapeDtypeStruct((T,), jnp.float32),
        in_specs=[
            pl.BlockSpec(memory_space=pltpu.VMEM),
            pl.BlockSpec(memory_space=pltpu.VMEM),
            pl.BlockSpec(memory_space=pltpu.VMEM),
        ],
        out_specs=pl.BlockSpec(memory_space=pltpu.VMEM),
        scratch_shapes=[
            pltpu.VMEM((8, T), jnp.float32),
            pltpu.VMEM((Y - 1, 8, T), jnp.float32),
            pltpu.SemaphoreType.DMA((Y - 1,)),
            pltpu.SemaphoreType.DMA((Y - 1,)),
        ],
        compiler_params=pltpu.CompilerParams(collective_id=0),
    )(x, W, labels2)


# baseline (device time: 34992 ns/iter reference)
import jax
import jax.numpy as jnp
from jax import lax
from jax.experimental import pallas as pl
from jax.experimental.pallas import tpu as pltpu

Y = 4
NEG = -1e30


def kernel(x, W, labels):
    T, D = x.shape
    Vs = W.shape[1]
    labels2 = labels.reshape(T, 1)

    def body(x_ref, w_ref, lab_ref, out_ref,
             stats_ref, comm_ref, send_sems, recv_sems):
        my_x = lax.axis_index("x")
        my_y = lax.axis_index("y")
        my_z = lax.axis_index("z")

        barrier = pltpu.get_barrier_semaphore()
        for d in range(1, Y):
            pl.semaphore_signal(
                barrier,
                inc=1,
                device_id=(my_x, (my_y + d) % Y, my_z),
                device_id_type=pl.DeviceIdType.MESH,
            )
        pl.semaphore_wait(barrier, Y - 1)

        xb = x_ref[...].astype(jnp.bfloat16)
        wb = w_ref[...].astype(jnp.bfloat16)
        logits = jnp.dot(xb, wb, preferred_element_type=jnp.float32)
        m = jnp.max(logits, axis=1)
        s = jnp.sum(jnp.exp(logits - m[:, None]), axis=1)
        cols = lax.broadcasted_iota(jnp.int32, (T, Vs), 1) + my_y * Vs
        mask = cols == lab_ref[...]
        lab = jnp.max(jnp.where(mask, logits, NEG), axis=1)

        stats_ref[0, :] = m
        stats_ref[1, :] = s
        stats_ref[2, :] = lab

        send_rdmas = []
        for d in range(1, Y):
            slot = Y - d - 1
            rdma = pltpu.make_async_remote_copy(
                src_ref=stats_ref,
                dst_ref=comm_ref.at[slot],
                send_sem=send_sems.at[d - 1],
                recv_sem=recv_sems.at[slot],
                device_id=(my_x, (my_y + d) % Y, my_z),
                device_id_type=pl.DeviceIdType.MESH,
            )
            rdma.start()
            send_rdmas.append(rdma)

        for k in range(Y - 1):
            pltpu.make_async_remote_copy(
                src_ref=stats_ref,
                dst_ref=comm_ref.at[k],
                send_sem=send_sems.at[0],
                recv_sem=recv_sems.at[k],
                device_id=(my_x, my_y, my_z),
                device_id_type=pl.DeviceIdType.MESH,
            ).wait_recv()

        gm = m
        for k in range(Y - 1):
            gm = jnp.maximum(gm, comm_ref[k, 0, :])
        stot = s * jnp.exp(m - gm)
        for k in range(Y - 1):
            stot = stot + comm_ref[k, 1, :] * jnp.exp(comm_ref[k, 0, :] - gm)
        glab = lab
        for k in range(Y - 1):
            glab = jnp.maximum(glab, comm_ref[k, 2, :])
        out_ref[...] = gm + jnp.log(stot) - glab

        for rdma in send_rdmas:
            rdma.wait_send()

    return pl.pallas_call(
        body,
        out_shape=jax.ShapeDtypeStruct((T,), jnp.float32),
        in_specs=[
            pl.BlockSpec(memory_space=pltpu.VMEM),
            pl.BlockSpec(memory_space=pltpu.VMEM),
            pl.BlockSpec(memory_space=pltpu.VMEM),
        ],
        out_specs=pl.BlockSpec(memory_space=pltpu.VMEM),
        scratch_shapes=[
            pltpu.VMEM((8, T), jnp.float32),
            pltpu.VMEM((Y - 1, 8, T), jnp.float32),
            pltpu.SemaphoreType.DMA((Y - 1,)),
            pltpu.SemaphoreType.DMA((Y - 1,)),
        ],
        compiler_params=pltpu.CompilerParams(
            collective_id=0, vmem_limit_bytes=100 * 1024 * 1024
        ),
    )(x, W, labels2)
